# baseline (speedup 1.0000x reference)
import functools

import jax
import jax.numpy as jnp
from jax import lax
from jax.experimental import pallas as pl
from jax.experimental.pallas import tpu as pltpu
from jax.experimental.pallas import tpu_sc as plsc

_INFO = plsc.get_sparse_core_info()
_NC = _INFO.num_cores
_NS = _INFO.num_subcores
_NW = _NC * _NS
_K = 128


@functools.partial(jax.jit, static_argnums=(0, 1, 2))
def _sc_aggregate(n_pad, n_edges, d, x, src, dst, zeros):
    epw = n_edges // _NW
    fc = epw // _K
    tail = epw - fc * _K
    bodies = fc // 2
    rows_per_tile = n_pad // _NS

    mesh = plsc.VectorSubcoreMesh(core_axis_name="c", subcore_axis_name="s")

    @functools.partial(
        pl.kernel,
        out_type=jax.ShapeDtypeStruct((_NC * n_pad, d), jnp.float32),
        mesh=mesh,
        scratch_types=[
            pltpu.VMEM((_K,), jnp.int32),
            pltpu.VMEM((_K,), jnp.int32),
            pltpu.VMEM((_K,), jnp.int32),
            pltpu.VMEM((_K,), jnp.int32),
            pltpu.VMEM((tail,), jnp.int32),
            pltpu.VMEM((tail,), jnp.int32),
            pltpu.VMEM((_K, d), jnp.float32),
            pltpu.VMEM((_K, d), jnp.float32),
            pltpu.VMEM((tail, d), jnp.float32),
            pltpu.VMEM_SHARED((n_pad, d), jnp.float32),
            pltpu.SemaphoreType.DMA,
            pltpu.SemaphoreType.DMA,
            pltpu.SemaphoreType.DMA,
            pltpu.SemaphoreType.DMA,
        ],
    )
    def agg_kernel(x_hbm, src_hbm, dst_hbm, zeros_hbm, part_hbm,
                   src_ia, src_ib, dst_ia, dst_ib, src_it, dst_it,
                   rows_a, rows_b, rows_t, acc_sh,
                   sem_ia, sem_ib, sem_a, sem_b):
        c = lax.axis_index("c")
        s = lax.axis_index("s")
        wid = s * _NC + c
        e0 = wid * epw

        def idx_load(i, src_i, dst_i, sem):
            pltpu.async_copy(src_hbm.at[pl.ds(e0 + i * _K, _K)], src_i, sem)
            pltpu.async_copy(dst_hbm.at[pl.ds(e0 + i * _K, _K)], dst_i, sem)

        def idx_wait(i, src_i, dst_i, sem):
            pltpu.make_async_copy(src_hbm.at[pl.ds(e0 + i * _K, _K)], src_i,
                                  sem).wait()
            pltpu.make_async_copy(dst_hbm.at[pl.ds(e0 + i * _K, _K)], dst_i,
                                  sem).wait()

        def gather(src_i, buf, sem):
            pltpu.async_copy(x_hbm.at[src_i], buf, sem)

        def gather_wait(src_i, buf, sem):
            pltpu.make_async_copy(x_hbm.at[src_i], buf, sem).wait()

        def scat(buf, dst_i):
            pltpu.sync_copy(buf, acc_sh.at[dst_i], add=True)

        idx_load(0, src_ia, dst_ia, sem_ia)
        idx_load(1, src_ib, dst_ib, sem_ib)
        pltpu.sync_copy(zeros_hbm,
                        acc_sh.at[pl.ds(s * rows_per_tile, rows_per_tile)])
        plsc.subcore_barrier()

        idx_wait(0, src_ia, dst_ia, sem_ia)
        gather(src_ia, rows_a, sem_a)

        def body(j, carry):
            i = 2 * j
            idx_wait(i + 1, src_ib, dst_ib, sem_ib)
            gather(src_ib, rows_b, sem_b)
            gather_wait(src_ia, rows_a, sem_a)
            scat(rows_a, dst_ia)

            @pl.when(j < bodies - 1)
            def _():
                idx_load(i + 2, src_ia, dst_ia, sem_ia)

            gather_wait(src_ib, rows_b, sem_b)
            scat(rows_b, dst_ib)

            @pl.when(j < bodies - 1)
            def _():
                idx_load(i + 3, src_ib, dst_ib, sem_ib)
                idx_wait(i + 2, src_ia, dst_ia, sem_ia)
                gather(src_ia, rows_a, sem_a)

            return carry

        lax.fori_loop(0, bodies, body, 0)

        cp_s = pltpu.async_copy(src_hbm.at[pl.ds(e0 + fc * _K, tail)],
                                src_it, sem_ia)
        cp_d = pltpu.async_copy(dst_hbm.at[pl.ds(e0 + fc * _K, tail)],
                                dst_it, sem_ia)
        cp_s.wait()
        cp_d.wait()
        pltpu.async_copy(x_hbm.at[src_it], rows_t, sem_a).wait()
        pltpu.sync_copy(rows_t, acc_sh.at[dst_it], add=True)
        plsc.subcore_barrier()

        off = c * n_pad + s * rows_per_tile
        pltpu.sync_copy(acc_sh.at[pl.ds(s * rows_per_tile, rows_per_tile)],
                        part_hbm.at[pl.ds(off, rows_per_tile)])

    return agg_kernel(x, src, dst, zeros)


def _tc_selfloop_body(x_ref, slw_ref, o_ref):
    o_ref[...] = lax.dot_general(
        x_ref[...], slw_ref[...], (((1,), (1,)), ((), ())),
        preferred_element_type=jnp.float32)


def _tc_combine_body(p0_ref, p1_ref, sl_ref, w_ref, o_ref):
    agg = p0_ref[0] + p1_ref[0]
    o_ref[...] = (
        jnp.dot(agg, w_ref[...], preferred_element_type=jnp.float32)
        + sl_ref[...]
    )


def kernel(x, edge_index, edge_type, weight, self_loop_w):
    n_nodes, d = x.shape
    n_edges = edge_index.shape[1]
    n_pad = ((n_nodes + 8 * _NS - 1) // (8 * _NS)) * (8 * _NS)
    zeros = jnp.zeros((n_pad // _NS, d), jnp.float32)

    blk = 2000
    grid = n_nodes // blk

    selfloop = pl.pallas_call(
        _tc_selfloop_body,
        grid=(grid,),
        in_specs=[
            pl.BlockSpec((blk, d), lambda i: (i, 0)),
            pl.BlockSpec((d, d), lambda i: (0, 0)),
        ],
        out_specs=pl.BlockSpec((blk, d), lambda i: (i, 0)),
        out_shape=jax.ShapeDtypeStruct((n_nodes, d), jnp.float32),
    )(x, self_loop_w)

    part = _sc_aggregate(n_pad, n_edges, d, x, edge_index[0], edge_index[1],
                         zeros)
    part3 = part.reshape(_NC, n_pad, d)

    out = pl.pallas_call(
        _tc_combine_body,
        grid=(grid,),
        in_specs=[
            pl.BlockSpec((1, blk, d), lambda i: (0, i, 0)),
            pl.BlockSpec((1, blk, d), lambda i: (1, i, 0)),
            pl.BlockSpec((blk, d), lambda i: (i, 0)),
            pl.BlockSpec((d, d), lambda i: (0, 0)),
        ],
        out_specs=pl.BlockSpec((blk, d), lambda i: (i, 0)),
        out_shape=jax.ShapeDtypeStruct((n_nodes, d), jnp.float32),
    )(part3, part3, selfloop, weight[0])
    return out


# speedup vs baseline: 1.0496x; 1.0496x over previous
import functools

import jax
import jax.numpy as jnp
from jax import lax
from jax.experimental import pallas as pl
from jax.experimental.pallas import tpu as pltpu
from jax.experimental.pallas import tpu_sc as plsc

_INFO = plsc.get_sparse_core_info()
_NC = _INFO.num_cores
_NS = _INFO.num_subcores
_NW = _NC * _NS
_K = 128


@functools.partial(jax.jit, static_argnums=(0, 1, 2))
def _sc_aggregate(n_pad, n_edges, d, x, src, dst, zeros):
    epw = n_edges // _NW
    fc = epw // _K
    tail = epw - fc * _K
    bodies = (fc - 2) // 4
    rem = fc - 4 * bodies
    rows_per_tile = n_pad // _NS

    mesh = plsc.VectorSubcoreMesh(core_axis_name="c", subcore_axis_name="s")

    @functools.partial(
        pl.kernel,
        out_type=jax.ShapeDtypeStruct((_NC * n_pad, d), jnp.float32),
        mesh=mesh,
        scratch_types=[
            pltpu.VMEM((4, _K), jnp.int32),
            pltpu.VMEM((4, _K), jnp.int32),
            pltpu.VMEM((tail,), jnp.int32),
            pltpu.VMEM((tail,), jnp.int32),
            pltpu.VMEM((_K, d), jnp.float32),
            pltpu.VMEM((_K, d), jnp.float32),
            pltpu.VMEM((tail, d), jnp.float32),
            pltpu.VMEM_SHARED((n_pad, d), jnp.float32),
            pltpu.SemaphoreType.DMA,
            pltpu.SemaphoreType.DMA,
            pltpu.SemaphoreType.DMA,
            pltpu.SemaphoreType.DMA,
            pltpu.SemaphoreType.DMA,
            pltpu.SemaphoreType.DMA,
        ],
    )
    def agg_kernel(x_hbm, src_hbm, dst_hbm, zeros_hbm, part_hbm,
                   src_i, dst_i, src_it, dst_it,
                   rows_a, rows_b, rows_t, acc_sh,
                   sem_i0, sem_i1, sem_i2, sem_i3, sem_a, sem_b):
        c = lax.axis_index("c")
        s = lax.axis_index("s")
        wid = s * _NC + c
        e0 = wid * epw
        sems = (sem_i0, sem_i1, sem_i2, sem_i3)

        def idx_load(i, slot):
            pltpu.async_copy(src_hbm.at[pl.ds(e0 + i * _K, _K)],
                             src_i.at[slot], sems[slot])
            pltpu.async_copy(dst_hbm.at[pl.ds(e0 + i * _K, _K)],
                             dst_i.at[slot], sems[slot])

        def idx_wait(i, slot):
            pltpu.make_async_copy(src_hbm.at[pl.ds(e0 + i * _K, _K)],
                                  src_i.at[slot], sems[slot]).wait()
            pltpu.make_async_copy(dst_hbm.at[pl.ds(e0 + i * _K, _K)],
                                  dst_i.at[slot], sems[slot]).wait()

        def gather(slot, buf, sem):
            pltpu.async_copy(x_hbm.at[src_i.at[slot]], buf, sem)

        def gather_wait(slot, buf, sem):
            pltpu.make_async_copy(x_hbm.at[src_i.at[slot]], buf, sem).wait()

        def scat(buf, slot):
            pltpu.sync_copy(buf, acc_sh.at[dst_i.at[slot]], add=True)

        for k in range(4):
            idx_load(k, k)
        pltpu.sync_copy(zeros_hbm,
                        acc_sh.at[pl.ds(s * rows_per_tile, rows_per_tile)])
        plsc.subcore_barrier()

        idx_wait(0, 0)
        gather(0, rows_a, sem_a)

        def body(j, carry):
            i = 4 * j
            idx_wait(i + 1, 1)
            gather(1, rows_b, sem_b)
            gather_wait(0, rows_a, sem_a)
            scat(rows_a, 0)
            idx_load(i + 4, 0)

            idx_wait(i + 2, 2)
            gather(2, rows_a, sem_a)
            gather_wait(1, rows_b, sem_b)
            scat(rows_b, 1)
            idx_load(i + 5, 1)

            idx_wait(i + 3, 3)
            gather(3, rows_b, sem_b)
            gather_wait(2, rows_a, sem_a)
            scat(rows_a, 2)

            @pl.when(i + 6 < fc)
            def _():
                idx_load(i + 6, 2)

            gather_wait(3, rows_b, sem_b)
            scat(rows_b, 3)

            @pl.when(i + 7 < fc)
            def _():
                idx_load(i + 7, 3)

            idx_wait(i + 4, 0)
            gather(0, rows_a, sem_a)
            return carry

        lax.fori_loop(0, bodies, body, 0)

        i0 = 4 * bodies
        idx_wait(i0 + 1, 1)
        gather(1, rows_b, sem_b)
        gather_wait(0, rows_a, sem_a)
        scat(rows_a, 0)
        gather_wait(1, rows_b, sem_b)
        scat(rows_b, 1)

        cp_s = pltpu.async_copy(src_hbm.at[pl.ds(e0 + fc * _K, tail)],
                                src_it, sem_i0)
        cp_d = pltpu.async_copy(dst_hbm.at[pl.ds(e0 + fc * _K, tail)],
                                dst_it, sem_i0)
        cp_s.wait()
        cp_d.wait()
        pltpu.async_copy(x_hbm.at[src_it], rows_t, sem_a).wait()
        pltpu.sync_copy(rows_t, acc_sh.at[dst_it], add=True)
        plsc.subcore_barrier()

        off = c * n_pad + s * rows_per_tile
        pltpu.sync_copy(acc_sh.at[pl.ds(s * rows_per_tile, rows_per_tile)],
                        part_hbm.at[pl.ds(off, rows_per_tile)])

    return agg_kernel(x, src, dst, zeros)


def _tc_selfloop_body(x_ref, slw_ref, o_ref):
    o_ref[...] = lax.dot_general(
        x_ref[...], slw_ref[...], (((1,), (1,)), ((), ())),
        preferred_element_type=jnp.float32)


def _tc_combine_body(p0_ref, p1_ref, sl_ref, w_ref, o_ref):
    agg = p0_ref[0] + p1_ref[0]
    o_ref[...] = (
        jnp.dot(agg, w_ref[...], preferred_element_type=jnp.float32)
        + sl_ref[...]
    )


def kernel(x, edge_index, edge_type, weight, self_loop_w):
    n_nodes, d = x.shape
    n_edges = edge_index.shape[1]
    n_pad = ((n_nodes + 8 * _NS - 1) // (8 * _NS)) * (8 * _NS)
    zeros = jnp.zeros((n_pad // _NS, d), jnp.float32)

    blk = 2000
    grid = n_nodes // blk

    selfloop = pl.pallas_call(
        _tc_selfloop_body,
        grid=(grid,),
        in_specs=[
            pl.BlockSpec((blk, d), lambda i: (i, 0)),
            pl.BlockSpec((d, d), lambda i: (0, 0)),
        ],
        out_specs=pl.BlockSpec((blk, d), lambda i: (i, 0)),
        out_shape=jax.ShapeDtypeStruct((n_nodes, d), jnp.float32),
    )(x, self_loop_w)

    part = _sc_aggregate(n_pad, n_edges, d, x, edge_index[0], edge_index[1],
                         zeros)
    part3 = part.reshape(_NC, n_pad, d)

    out = pl.pallas_call(
        _tc_combine_body,
        grid=(grid,),
        in_specs=[
            pl.BlockSpec((1, blk, d), lambda i: (0, i, 0)),
            pl.BlockSpec((1, blk, d), lambda i: (1, i, 0)),
            pl.BlockSpec((blk, d), lambda i: (i, 0)),
            pl.BlockSpec((d, d), lambda i: (0, 0)),
        ],
        out_specs=pl.BlockSpec((blk, d), lambda i: (i, 0)),
        out_shape=jax.ShapeDtypeStruct((n_nodes, d), jnp.float32),
    )(part3, part3, selfloop, weight[0])
    return out


# speedup vs baseline: 1.2168x; 1.1593x over previous
import functools

import jax
import jax.numpy as jnp
from jax import lax
from jax.experimental import pallas as pl
from jax.experimental.pallas import tpu as pltpu
from jax.experimental.pallas import tpu_sc as plsc

_INFO = plsc.get_sparse_core_info()
_NC = _INFO.num_cores
_NS = _INFO.num_subcores
_NW = _NC * _NS
_K = 125


@functools.partial(jax.jit, static_argnums=(0, 1, 2))
def _sc_aggregate(n_pad, n_edges, d, x, eidx3, zeros):
    edges_per_worker = n_edges // _NW
    steps = edges_per_worker // _K
    n_phases = 5
    hs = steps // n_phases
    rows_per_tile = n_pad // _NS

    mesh = plsc.VectorSubcoreMesh(core_axis_name="c", subcore_axis_name="s")

    @functools.partial(
        pl.kernel,
        out_type=jax.ShapeDtypeStruct((_NC * n_pad, d), jnp.float32),
        mesh=mesh,
        scratch_types=[
            pltpu.VMEM((2, hs, _K), jnp.int32),
            pltpu.VMEM((2, hs, _K), jnp.int32),
            pltpu.VMEM((_K, d), jnp.float32),
            pltpu.VMEM((_K, d), jnp.float32),
            pltpu.VMEM_SHARED((n_pad, d), jnp.float32),
            pltpu.SemaphoreType.DMA,
            pltpu.SemaphoreType.DMA,
            pltpu.SemaphoreType.DMA,
        ],
    )
    def agg_kernel(x_hbm, eidx_hbm, zeros_hbm, part_hbm,
                   src_v, dst_v, rows_a, rows_b, acc_sh,
                   sem_i, sem_a, sem_b):
        c = lax.axis_index("c")
        s = lax.axis_index("s")
        wid = s * _NC + c
        row0 = wid * steps

        def load_idx(p, slot):
            r0 = row0 + p * hs
            pltpu.async_copy(eidx_hbm.at[0, pl.ds(r0, hs)], src_v.at[slot],
                             sem_i)
            pltpu.async_copy(eidx_hbm.at[1, pl.ds(r0, hs)], dst_v.at[slot],
                             sem_i)

        def wait_idx(p, slot):
            r0 = row0 + p * hs
            pltpu.make_async_copy(eidx_hbm.at[0, pl.ds(r0, hs)],
                                  src_v.at[slot], sem_i).wait()
            pltpu.make_async_copy(eidx_hbm.at[1, pl.ds(r0, hs)],
                                  dst_v.at[slot], sem_i).wait()

        load_idx(0, 0)
        pltpu.sync_copy(zeros_hbm,
                        acc_sh.at[pl.ds(s * rows_per_tile, rows_per_tile)])
        wait_idx(0, 0)
        plsc.subcore_barrier()

        for p in range(n_phases):
            slot = p % 2
            if p + 1 < n_phases:
                load_idx(p + 1, 1 - slot)

            def gather(i, buf, sem):
                return pltpu.async_copy(x_hbm.at[src_v.at[slot, i]], buf, sem)

            def scat(i, buf):
                pltpu.sync_copy(buf, acc_sh.at[dst_v.at[slot, i]], add=True)

            gather(0, rows_a, sem_a)

            def body(j, carry):
                i = 2 * j
                gather(i + 1, rows_b, sem_b)
                pltpu.make_async_copy(x_hbm.at[src_v.at[slot, i]], rows_a,
                                      sem_a).wait()
                scat(i, rows_a)

                @pl.when(j < hs // 2 - 1)
                def _():
                    gather(i + 2, rows_a, sem_a)

                pltpu.make_async_copy(x_hbm.at[src_v.at[slot, i + 1]], rows_b,
                                      sem_b).wait()
                scat(i + 1, rows_b)
                return carry

            lax.fori_loop(0, hs // 2, body, 0)
            if p + 1 < n_phases:
                wait_idx(p + 1, 1 - slot)
        plsc.subcore_barrier()

        off = c * n_pad + s * rows_per_tile
        pltpu.sync_copy(acc_sh.at[pl.ds(s * rows_per_tile, rows_per_tile)],
                        part_hbm.at[pl.ds(off, rows_per_tile)])

    return agg_kernel(x, eidx3, zeros)


def _tc_selfloop_body(x_ref, slw_ref, o_ref):
    o_ref[...] = lax.dot_general(
        x_ref[...], slw_ref[...], (((1,), (1,)), ((), ())),
        preferred_element_type=jnp.float32)


def _tc_combine_body(p0_ref, p1_ref, sl_ref, w_ref, o_ref):
    agg = p0_ref[0] + p1_ref[0]
    o_ref[...] = (
        jnp.dot(agg, w_ref[...], preferred_element_type=jnp.float32)
        + sl_ref[...]
    )


def kernel(x, edge_index, edge_type, weight, self_loop_w):
    n_nodes, d = x.shape
    n_edges = edge_index.shape[1]
    n_pad = ((n_nodes + 8 * _NS - 1) // (8 * _NS)) * (8 * _NS)
    eidx3 = edge_index.reshape(2, n_edges // _K, _K)
    zeros = jnp.zeros((n_pad // _NS, d), jnp.float32)

    blk = 2000
    grid = n_nodes // blk

    selfloop = pl.pallas_call(
        _tc_selfloop_body,
        grid=(grid,),
        in_specs=[
            pl.BlockSpec((blk, d), lambda i: (i, 0)),
            pl.BlockSpec((d, d), lambda i: (0, 0)),
        ],
        out_specs=pl.BlockSpec((blk, d), lambda i: (i, 0)),
        out_shape=jax.ShapeDtypeStruct((n_nodes, d), jnp.float32),
    )(x, self_loop_w)

    part = _sc_aggregate(n_pad, n_edges, d, x, eidx3, zeros)
    part3 = part.reshape(_NC, n_pad, d)

    out = pl.pallas_call(
        _tc_combine_body,
        grid=(grid,),
        in_specs=[
            pl.BlockSpec((1, blk, d), lambda i: (0, i, 0)),
            pl.BlockSpec((1, blk, d), lambda i: (1, i, 0)),
            pl.BlockSpec((blk, d), lambda i: (i, 0)),
            pl.BlockSpec((d, d), lambda i: (0, 0)),
        ],
        out_specs=pl.BlockSpec((blk, d), lambda i: (i, 0)),
        out_shape=jax.ShapeDtypeStruct((n_nodes, d), jnp.float32),
    )(part3, part3, selfloop, weight[0])
    return out


# speedup vs baseline: 1.2554x; 1.0317x over previous
import functools

import jax
import jax.numpy as jnp
from jax import lax
from jax.experimental import pallas as pl
from jax.experimental.pallas import tpu as pltpu
from jax.experimental.pallas import tpu_sc as plsc

_INFO = plsc.get_sparse_core_info()
_NC = _INFO.num_cores
_NS = _INFO.num_subcores
_NW = _NC * _NS
_K = 125


@functools.partial(jax.jit, static_argnums=(0, 1, 2))
def _sc_aggregate(n_pad, n_edges, d, x, eidx3, zeros):
    edges_per_worker = n_edges // _NW
    steps = edges_per_worker // _K
    n_phases = 2
    hs = steps // n_phases
    rows_per_tile = n_pad // _NS

    mesh = plsc.VectorSubcoreMesh(core_axis_name="c", subcore_axis_name="s")

    @functools.partial(
        pl.kernel,
        out_type=jax.ShapeDtypeStruct((_NC * n_pad, d), jnp.float32),
        mesh=mesh,
        scratch_types=[
            pltpu.VMEM((hs, _K), jnp.int32),
            pltpu.VMEM((hs, _K), jnp.int32),
            pltpu.VMEM((_K, d), jnp.float32),
            pltpu.VMEM((_K, d), jnp.float32),
            pltpu.VMEM_SHARED((n_pad, d), jnp.float32),
            pltpu.SemaphoreType.DMA,
            pltpu.SemaphoreType.DMA,
            pltpu.SemaphoreType.DMA,
        ],
    )
    def agg_kernel(x_hbm, eidx_hbm, zeros_hbm, part_hbm,
                   src_v, dst_v, rows_a, rows_b, acc_sh,
                   sem_i, sem_a, sem_b):
        c = lax.axis_index("c")
        s = lax.axis_index("s")
        wid = s * _NC + c
        row0 = wid * steps

        def gather(i, buf, sem):
            return pltpu.async_copy(x_hbm.at[src_v.at[i]], buf, sem)

        def scat(i, buf):
            pltpu.sync_copy(buf, acc_sh.at[dst_v.at[i]], add=True)

        for p in range(n_phases):
            r0 = row0 + p * hs
            cp_src = pltpu.async_copy(eidx_hbm.at[0, pl.ds(r0, hs)], src_v,
                                      sem_i)
            cp_dst = pltpu.async_copy(eidx_hbm.at[1, pl.ds(r0, hs)], dst_v,
                                      sem_i)
            if p == 0:
                pltpu.sync_copy(
                    zeros_hbm,
                    acc_sh.at[pl.ds(s * rows_per_tile, rows_per_tile)])
            cp_src.wait()
            cp_dst.wait()
            if p == 0:
                plsc.subcore_barrier()

            gather(0, rows_a, sem_a)

            def body(j, carry):
                i = 2 * j
                gather(i + 1, rows_b, sem_b)
                pltpu.make_async_copy(x_hbm.at[src_v.at[i]], rows_a,
                                      sem_a).wait()
                scat(i, rows_a)

                @pl.when(j < hs // 2 - 1)
                def _():
                    gather(i + 2, rows_a, sem_a)

                pltpu.make_async_copy(x_hbm.at[src_v.at[i + 1]], rows_b,
                                      sem_b).wait()
                scat(i + 1, rows_b)
                return carry

            lax.fori_loop(0, hs // 2, body, 0)
        plsc.subcore_barrier()

        off = c * n_pad + s * rows_per_tile
        pltpu.sync_copy(acc_sh.at[pl.ds(s * rows_per_tile, rows_per_tile)],
                        part_hbm.at[pl.ds(off, rows_per_tile)])

    return agg_kernel(x, eidx3, zeros)


def _tc_selfloop_body(x_ref, slw_ref, o_ref):
    o_ref[...] = lax.dot_general(
        x_ref[...], slw_ref[...], (((1,), (1,)), ((), ())),
        preferred_element_type=jnp.float32)


def _tc_combine_body(p0_ref, p1_ref, sl_ref, w_ref, o_ref):
    agg = p0_ref[0] + p1_ref[0]
    o_ref[...] = (
        jnp.dot(agg, w_ref[...], preferred_element_type=jnp.float32)
        + sl_ref[...]
    )


def kernel(x, edge_index, edge_type, weight, self_loop_w):
    n_nodes, d = x.shape
    n_edges = edge_index.shape[1]
    n_pad = ((n_nodes + 8 * _NS - 1) // (8 * _NS)) * (8 * _NS)
    eidx3 = edge_index.reshape(2, n_edges // _K, _K)
    zeros = jnp.zeros((n_pad // _NS, d), jnp.float32)

    blk = 2000
    grid = n_nodes // blk

    selfloop = pl.pallas_call(
        _tc_selfloop_body,
        grid=(grid,),
        in_specs=[
            pl.BlockSpec((blk, d), lambda i: (i, 0)),
            pl.BlockSpec((d, d), lambda i: (0, 0)),
        ],
        out_specs=pl.BlockSpec((blk, d), lambda i: (i, 0)),
        out_shape=jax.ShapeDtypeStruct((n_nodes, d), jnp.float32),
    )(x, self_loop_w)

    part = _sc_aggregate(n_pad, n_edges, d, x, eidx3, zeros)
    part3 = part.reshape(_NC, n_pad, d)

    out = pl.pallas_call(
        _tc_combine_body,
        grid=(grid,),
        in_specs=[
            pl.BlockSpec((1, blk, d), lambda i: (0, i, 0)),
            pl.BlockSpec((1, blk, d), lambda i: (1, i, 0)),
            pl.BlockSpec((blk, d), lambda i: (i, 0)),
            pl.BlockSpec((d, d), lambda i: (0, 0)),
        ],
        out_specs=pl.BlockSpec((blk, d), lambda i: (i, 0)),
        out_shape=jax.ShapeDtypeStruct((n_nodes, d), jnp.float32),
    )(part3, part3, selfloop, weight[0])
    return out
